# Initial kernel scaffold; baseline (speedup 1.0000x reference)
import jax
import jax.numpy as jnp
from jax.experimental import pallas as pl


def kernel(x, edge_index, edge_weigth, batch, W1, b1, W2, b2, W3, b3, W4, b4, gamma, beta, fcw1, fcb1, fcw2, fcb2, fcw3, fcb3):
    raise NotImplementedError("write your pallas kernel here")


# speedup vs baseline: 3.7119x; 3.7119x over previous
import functools

import jax
import jax.numpy as jnp
from jax import lax
from jax.experimental import pallas as pl
from jax.experimental.pallas import tpu as pltpu
from jax.experimental.pallas import tpu_sc as plsc

N = 10000
E = 320000
G = 256

NC = 2
NS = 16
NW = NC * NS
EPW = E // NW
C = 80
NCHUNK = EPW // C
RPT = 624
RTAIL = N - NS * RPT

BR = 1000


def _leaky(t):
    return jnp.maximum(t, 0.01 * t)


@functools.cache
def _sc_scatter(F):
    mesh = plsc.VectorSubcoreMesh(core_axis_name="c", subcore_axis_name="s")

    @functools.partial(
        pl.kernel,
        out_type=jax.ShapeDtypeStruct((2 * N, F), jnp.float32),
        mesh=mesh,
        scratch_types=[
            pltpu.VMEM((C,), jnp.int32),
            pltpu.VMEM((C,), jnp.int32),
            pltpu.VMEM((C,), jnp.float32),
            pltpu.VMEM((C, F), jnp.float32),
            pltpu.VMEM_SHARED((N, F), jnp.float32),
            pltpu.SemaphoreType.DMA,
        ],
        compiler_params=pltpu.CompilerParams(use_tc_tiling_on_sc=False),
    )
    def scat(h_hbm, src_hbm, dst_hbm, ew_hbm, zero_hbm, out_hbm,
             src_v, dst_v, ew_v, rows_v, acc_s, sem):
        cid = lax.axis_index("c")
        sid = lax.axis_index("s")
        wid = sid * NC + cid
        iota16 = lax.broadcasted_iota(jnp.int32, (16,), 0)

        pltpu.sync_copy(zero_hbm.at[pl.ds(sid * RPT, RPT)],
                        acc_s.at[pl.ds(sid * RPT, RPT)])

        @pl.when(sid == NS - 1)
        def _():
            pltpu.sync_copy(zero_hbm.at[pl.ds(NS * RPT, RTAIL)],
                            acc_s.at[pl.ds(NS * RPT, RTAIL)])

        plsc.subcore_barrier()

        ebase = wid * EPW

        def chunk(c, carry):
            base = ebase + c * C
            pltpu.sync_copy(src_hbm.at[pl.ds(base, C)], src_v)
            pltpu.sync_copy(dst_hbm.at[pl.ds(base, C)], dst_v)
            pltpu.sync_copy(ew_hbm.at[pl.ds(base, C)], ew_v)
            pltpu.async_copy(h_hbm.at[src_v], rows_v, sem).wait()

            def group(g, carry2):
                ew16 = ew_v[pl.ds(g * 16, 16)]
                for e in range(16):
                    w = ew16.at[jnp.full((16,), e, jnp.int32)].get(
                        mode="promise_in_bounds")
                    r = g * 16 + e
                    for f in range(F // 16):
                        v = rows_v[r, pl.ds(f * 16, 16)]
                        rows_v[r, pl.ds(f * 16, 16)] = v * w
                return carry2

            lax.fori_loop(0, C // 16, group, 0)
            pltpu.sync_copy(rows_v, acc_s.at[dst_v], add=True)
            return carry

        lax.fori_loop(0, NCHUNK, chunk, 0)
        plsc.subcore_barrier()
        pltpu.sync_copy(acc_s.at[pl.ds(sid * RPT, RPT)],
                        out_hbm.at[pl.ds(cid * N + sid * RPT, RPT)])

        @pl.when(sid == NS - 1)
        def _():
            pltpu.sync_copy(acc_s.at[pl.ds(NS * RPT, RTAIL)],
                            out_hbm.at[pl.ds(cid * N + NS * RPT, RTAIL)])

    return scat


def _mm_first(x, W):
    def body(x_ref, w_ref, o_ref):
        o_ref[...] = jnp.dot(x_ref[...], w_ref[...],
                             preferred_element_type=jnp.float32)

    Fi, Fo = W.shape
    return pl.pallas_call(
        body,
        grid=(N // BR,),
        in_specs=[pl.BlockSpec((BR, Fi), lambda i: (i, 0)),
                  pl.BlockSpec((Fi, Fo), lambda i: (0, 0))],
        out_specs=pl.BlockSpec((BR, Fo), lambda i: (i, 0)),
        out_shape=jax.ShapeDtypeStruct((N, Fo), jnp.float32),
    )(x, W)


def _fuse_layer(p, b, W):
    def body(p_ref, b_ref, w_ref, o_ref):
        s = p_ref[0] + p_ref[1] + b_ref[...]
        o_ref[...] = jnp.dot(_leaky(s), w_ref[...],
                             preferred_element_type=jnp.float32)

    F = p.shape[-1]
    Fo = W.shape[1]
    return pl.pallas_call(
        body,
        grid=(N // BR,),
        in_specs=[pl.BlockSpec((2, BR, F), lambda i: (0, i, 0)),
                  pl.BlockSpec((1, F), lambda i: (0, 0)),
                  pl.BlockSpec((F, Fo), lambda i: (0, 0))],
        out_specs=pl.BlockSpec((BR, Fo), lambda i: (i, 0)),
        out_shape=jax.ShapeDtypeStruct((N, Fo), jnp.float32),
    )(p, b, W)


def _final_conv(p, b):
    def body(p_ref, b_ref, conv_ref, st_ref):
        s = p_ref[0] + p_ref[1] + b_ref[...]
        conv_ref[...] = s

        @pl.when(pl.program_id(0) == 0)
        def _():
            st_ref[...] = jnp.zeros_like(st_ref)

        st_ref[0:1, :] = st_ref[0:1, :] + jnp.sum(s, axis=0, keepdims=True)
        st_ref[1:2, :] = st_ref[1:2, :] + jnp.sum(s * s, axis=0,
                                                  keepdims=True)

    F = p.shape[-1]
    return pl.pallas_call(
        body,
        grid=(N // BR,),
        in_specs=[pl.BlockSpec((2, BR, F), lambda i: (0, i, 0)),
                  pl.BlockSpec((1, F), lambda i: (0, 0))],
        out_specs=[pl.BlockSpec((BR, F), lambda i: (i, 0)),
                   pl.BlockSpec((8, F), lambda i: (0, 0))],
        out_shape=[jax.ShapeDtypeStruct((N, F), jnp.float32),
                   jax.ShapeDtypeStruct((8, F), jnp.float32)],
    )(p, b)


def _bn_pool(conv, stats, gamma, beta, batch3d):
    def body(c_ref, st_ref, g_ref, b_ref, bt_ref, o_ref):
        mean = st_ref[0:1, :] * (1.0 / N)
        var = st_ref[1:2, :] * (1.0 / N) - mean * mean
        inv = lax.rsqrt(var + 1e-5)
        s = (c_ref[...] - mean) * inv * g_ref[...] + b_ref[...]
        s = _leaky(s)
        sel = (bt_ref[0] ==
               lax.broadcasted_iota(jnp.int32, (G, 1), 0)).astype(jnp.float32)
        part = jnp.dot(sel, s, preferred_element_type=jnp.float32)

        @pl.when(pl.program_id(0) == 0)
        def _():
            o_ref[...] = jnp.zeros_like(o_ref)

        o_ref[...] = o_ref[...] + part

    F = conv.shape[-1]
    return pl.pallas_call(
        body,
        grid=(N // BR,),
        in_specs=[pl.BlockSpec((BR, F), lambda i: (i, 0)),
                  pl.BlockSpec((8, F), lambda i: (0, 0)),
                  pl.BlockSpec((1, F), lambda i: (0, 0)),
                  pl.BlockSpec((1, F), lambda i: (0, 0)),
                  pl.BlockSpec((1, 1, BR), lambda i: (i, 0, 0))],
        out_specs=pl.BlockSpec((G, F), lambda i: (0, 0)),
        out_shape=jax.ShapeDtypeStruct((G, F), jnp.float32),
    )(conv, stats, gamma, beta, batch3d)


def _mlp(pooled, w1, b1, w2, b2, w3, b3):
    def body(p_ref, w1r, b1r, w2r, b2r, w3r, b3r, o_ref):
        a = _leaky(jnp.dot(p_ref[...], w1r[...],
                           preferred_element_type=jnp.float32) + b1r[...])
        a = _leaky(jnp.dot(a, w2r[...],
                           preferred_element_type=jnp.float32) + b2r[...])
        a = _leaky(jnp.dot(a, w3r[...],
                           preferred_element_type=jnp.float32) + b3r[...])
        o_ref[...] = a

    H = w1.shape[1]
    return pl.pallas_call(
        body,
        in_specs=[pl.BlockSpec(pooled.shape, lambda: (0, 0)),
                  pl.BlockSpec(w1.shape, lambda: (0, 0)),
                  pl.BlockSpec(b1.shape, lambda: (0, 0)),
                  pl.BlockSpec(w2.shape, lambda: (0, 0)),
                  pl.BlockSpec(b2.shape, lambda: (0, 0)),
                  pl.BlockSpec(w3.shape, lambda: (0, 0)),
                  pl.BlockSpec(b3.shape, lambda: (0, 0))],
        out_specs=pl.BlockSpec((G, H), lambda: (0, 0)),
        out_shape=jax.ShapeDtypeStruct((G, H), jnp.float32),
    )(pooled, w1, b1, w2, b2, w3, b3)


def _pad2(a, r, c):
    return jnp.pad(a, ((0, r - a.shape[0]), (0, c - a.shape[1])))


def kernel(x, edge_index, edge_weigth, batch, W1, b1, W2, b2, W3, b3, W4, b4,
           gamma, beta, fcw1, fcb1, fcw2, fcb2, fcw3, fcb3):
    src = edge_index[0]
    dst = edge_index[1]

    W4p = _pad2(W4, 64, 64)
    b4p = jnp.pad(b4, (0, 14)).reshape(1, 64)
    gammap = jnp.pad(gamma, (0, 14)).reshape(1, 64)
    betap = jnp.pad(beta, (0, 14)).reshape(1, 64)
    fw1 = _pad2(fcw1, 64, 128)
    fb1 = jnp.pad(fcb1, (0, 98)).reshape(1, 128)
    fw2 = _pad2(fcw2, 128, 128)
    fb2 = jnp.pad(fcb2, (0, 108)).reshape(1, 128)
    fw3 = _pad2(fcw3, 128, 128)
    fb3 = jnp.pad(fcb3, (0, 126)).reshape(1, 128)

    def scat(h, F):
        zeros = jnp.zeros((N, F), jnp.float32)
        p = _sc_scatter(F)(h, src, dst, edge_weigth, zeros)
        return p.reshape(2, N, F)

    h1 = _mm_first(x, W1)
    p1 = scat(h1, 16)
    h2 = _fuse_layer(p1, b1.reshape(1, 16), W2)
    p2 = scat(h2, 32)
    h3 = _fuse_layer(p2, b2.reshape(1, 32), W3)
    p3 = scat(h3, 64)
    h4 = _fuse_layer(p3, b3.reshape(1, 64), W4p)
    p4 = scat(h4, 64)
    conv, stats = _final_conv(p4, b4p)
    pooled = _bn_pool(conv, stats, gammap, betap, batch.reshape(N // BR, 1, BR))
    out = _mlp(pooled, fw1, fb1, fw2, fb2, fw3, fb3)
    return out[:, :2]


# speedup vs baseline: 7.9884x; 2.1521x over previous
import functools

import jax
import jax.numpy as jnp
from jax import lax
from jax.experimental import pallas as pl
from jax.experimental.pallas import tpu as pltpu
from jax.experimental.pallas import tpu_sc as plsc

N = 10000
E = 320000
G = 256

NC = 2
NS = 16
NW = NC * NS
EPW = E // NW
C = 80
NCHUNK = EPW // C
K = 5
NSUPER = NCHUNK // K
RPT = 624
RTAIL = N - NS * RPT

BR = 1000


def _leaky(t):
    return jnp.maximum(t, 0.01 * t)


@functools.cache
def _sc_scatter(F):
    mesh = plsc.VectorSubcoreMesh(core_axis_name="c", subcore_axis_name="s")

    @functools.partial(
        pl.kernel,
        out_type=jax.ShapeDtypeStruct((2 * N, F), jnp.float32),
        mesh=mesh,
        scratch_types=(
            [pltpu.VMEM((NCHUNK, C), jnp.int32),
             pltpu.VMEM((NCHUNK, C), jnp.int32),
             pltpu.VMEM((NCHUNK, C), jnp.float32)]
            + [pltpu.VMEM((C, F), jnp.float32) for _ in range(K)]
            + [pltpu.VMEM_SHARED((N, F), jnp.float32),
               pltpu.SemaphoreType.DMA,
               pltpu.SemaphoreType.DMA]
        ),
        compiler_params=pltpu.CompilerParams(use_tc_tiling_on_sc=False),
    )
    def scat(h_hbm, src_hbm, dst_hbm, ew_hbm, zero_hbm, out_hbm, *refs):
        src_v, dst_v, ew_v = refs[0], refs[1], refs[2]
        rows = refs[3:3 + K]
        acc_s, gsem, ssem = refs[3 + K], refs[4 + K], refs[5 + K]
        cid = lax.axis_index("c")
        sid = lax.axis_index("s")
        wid = sid * NC + cid

        pltpu.sync_copy(zero_hbm.at[pl.ds(sid * RPT, RPT)],
                        acc_s.at[pl.ds(sid * RPT, RPT)])

        @pl.when(sid == NS - 1)
        def _():
            pltpu.sync_copy(zero_hbm.at[pl.ds(NS * RPT, RTAIL)],
                            acc_s.at[pl.ds(NS * RPT, RTAIL)])

        cbase0 = wid * NCHUNK
        pltpu.sync_copy(src_hbm.at[pl.ds(cbase0, NCHUNK)], src_v)
        pltpu.sync_copy(dst_hbm.at[pl.ds(cbase0, NCHUNK)], dst_v)
        pltpu.sync_copy(ew_hbm.at[pl.ds(cbase0, NCHUNK)], ew_v)
        plsc.subcore_barrier()

        def batch(s, carry):
            cb = s * K
            gd = [pltpu.async_copy(h_hbm.at[src_v.at[cb + j]], rows[j], gsem)
                  for j in range(K)]
            for d in gd:
                d.wait()
            for j in range(K):
                def group(g, carry2, j=j):
                    ew16 = ew_v[cb + j, pl.ds(g * 16, 16)]
                    for e in range(16):
                        w = ew16.at[jnp.full((16,), e, jnp.int32)].get(
                            mode="promise_in_bounds")
                        r = g * 16 + e
                        for f in range(F // 16):
                            v = rows[j][r, pl.ds(f * 16, 16)]
                            rows[j][r, pl.ds(f * 16, 16)] = v * w
                    return carry2

                lax.fori_loop(0, C // 16, group, 0)
            sd = [pltpu.async_copy(rows[j], acc_s.at[dst_v.at[cb + j]], ssem,
                                   add=True)
                  for j in range(K)]
            for d in sd:
                d.wait()
            return carry

        lax.fori_loop(0, NSUPER, batch, 0)
        plsc.subcore_barrier()
        pltpu.sync_copy(acc_s.at[pl.ds(sid * RPT, RPT)],
                        out_hbm.at[pl.ds(cid * N + sid * RPT, RPT)])

        @pl.when(sid == NS - 1)
        def _():
            pltpu.sync_copy(acc_s.at[pl.ds(NS * RPT, RTAIL)],
                            out_hbm.at[pl.ds(cid * N + NS * RPT, RTAIL)])

    return scat


def _mm_first(x, W):
    def body(x_ref, w_ref, o_ref):
        o_ref[...] = jnp.dot(x_ref[...], w_ref[...],
                             preferred_element_type=jnp.float32)

    Fi, Fo = W.shape
    return pl.pallas_call(
        body,
        grid=(N // BR,),
        in_specs=[pl.BlockSpec((BR, Fi), lambda i: (i, 0)),
                  pl.BlockSpec((Fi, Fo), lambda i: (0, 0))],
        out_specs=pl.BlockSpec((BR, Fo), lambda i: (i, 0)),
        out_shape=jax.ShapeDtypeStruct((N, Fo), jnp.float32),
    )(x, W)


def _fuse_layer(p, b, W):
    def body(p_ref, b_ref, w_ref, o_ref):
        s = p_ref[0] + p_ref[1] + b_ref[...]
        o_ref[...] = jnp.dot(_leaky(s), w_ref[...],
                             preferred_element_type=jnp.float32)

    F = p.shape[-1]
    Fo = W.shape[1]
    return pl.pallas_call(
        body,
        grid=(N // BR,),
        in_specs=[pl.BlockSpec((2, BR, F), lambda i: (0, i, 0)),
                  pl.BlockSpec((1, F), lambda i: (0, 0)),
                  pl.BlockSpec((F, Fo), lambda i: (0, 0))],
        out_specs=pl.BlockSpec((BR, Fo), lambda i: (i, 0)),
        out_shape=jax.ShapeDtypeStruct((N, Fo), jnp.float32),
    )(p, b, W)


def _final_conv(p, b):
    def body(p_ref, b_ref, conv_ref, st_ref):
        s = p_ref[0] + p_ref[1] + b_ref[...]
        conv_ref[...] = s

        @pl.when(pl.program_id(0) == 0)
        def _():
            st_ref[...] = jnp.zeros_like(st_ref)

        st_ref[0:1, :] = st_ref[0:1, :] + jnp.sum(s, axis=0, keepdims=True)
        st_ref[1:2, :] = st_ref[1:2, :] + jnp.sum(s * s, axis=0,
                                                  keepdims=True)

    F = p.shape[-1]
    return pl.pallas_call(
        body,
        grid=(N // BR,),
        in_specs=[pl.BlockSpec((2, BR, F), lambda i: (0, i, 0)),
                  pl.BlockSpec((1, F), lambda i: (0, 0))],
        out_specs=[pl.BlockSpec((BR, F), lambda i: (i, 0)),
                   pl.BlockSpec((8, F), lambda i: (0, 0))],
        out_shape=[jax.ShapeDtypeStruct((N, F), jnp.float32),
                   jax.ShapeDtypeStruct((8, F), jnp.float32)],
    )(p, b)


def _bn_pool(conv, stats, gamma, beta, batch3d):
    def body(c_ref, st_ref, g_ref, b_ref, bt_ref, o_ref):
        mean = st_ref[0:1, :] * (1.0 / N)
        var = st_ref[1:2, :] * (1.0 / N) - mean * mean
        inv = lax.rsqrt(var + 1e-5)
        s = (c_ref[...] - mean) * inv * g_ref[...] + b_ref[...]
        s = _leaky(s)
        sel = (bt_ref[0] ==
               lax.broadcasted_iota(jnp.int32, (G, 1), 0)).astype(jnp.float32)
        part = jnp.dot(sel, s, preferred_element_type=jnp.float32)

        @pl.when(pl.program_id(0) == 0)
        def _():
            o_ref[...] = jnp.zeros_like(o_ref)

        o_ref[...] = o_ref[...] + part

    F = conv.shape[-1]
    return pl.pallas_call(
        body,
        grid=(N // BR,),
        in_specs=[pl.BlockSpec((BR, F), lambda i: (i, 0)),
                  pl.BlockSpec((8, F), lambda i: (0, 0)),
                  pl.BlockSpec((1, F), lambda i: (0, 0)),
                  pl.BlockSpec((1, F), lambda i: (0, 0)),
                  pl.BlockSpec((1, 1, BR), lambda i: (i, 0, 0))],
        out_specs=pl.BlockSpec((G, F), lambda i: (0, 0)),
        out_shape=jax.ShapeDtypeStruct((G, F), jnp.float32),
    )(conv, stats, gamma, beta, batch3d)


def _mlp(pooled, w1, b1, w2, b2, w3, b3):
    def body(p_ref, w1r, b1r, w2r, b2r, w3r, b3r, o_ref):
        a = _leaky(jnp.dot(p_ref[...], w1r[...],
                           preferred_element_type=jnp.float32) + b1r[...])
        a = _leaky(jnp.dot(a, w2r[...],
                           preferred_element_type=jnp.float32) + b2r[...])
        a = _leaky(jnp.dot(a, w3r[...],
                           preferred_element_type=jnp.float32) + b3r[...])
        o_ref[...] = a

    H = w1.shape[1]
    return pl.pallas_call(
        body,
        in_specs=[pl.BlockSpec(pooled.shape, lambda: (0, 0)),
                  pl.BlockSpec(w1.shape, lambda: (0, 0)),
                  pl.BlockSpec(b1.shape, lambda: (0, 0)),
                  pl.BlockSpec(w2.shape, lambda: (0, 0)),
                  pl.BlockSpec(b2.shape, lambda: (0, 0)),
                  pl.BlockSpec(w3.shape, lambda: (0, 0)),
                  pl.BlockSpec(b3.shape, lambda: (0, 0))],
        out_specs=pl.BlockSpec((G, H), lambda: (0, 0)),
        out_shape=jax.ShapeDtypeStruct((G, H), jnp.float32),
    )(pooled, w1, b1, w2, b2, w3, b3)


def _pad2(a, r, c):
    return jnp.pad(a, ((0, r - a.shape[0]), (0, c - a.shape[1])))


def kernel(x, edge_index, edge_weigth, batch, W1, b1, W2, b2, W3, b3, W4, b4,
           gamma, beta, fcw1, fcb1, fcw2, fcb2, fcw3, fcb3):
    src = edge_index[0]
    dst = edge_index[1]

    W4p = _pad2(W4, 64, 64)
    b4p = jnp.pad(b4, (0, 14)).reshape(1, 64)
    gammap = jnp.pad(gamma, (0, 14)).reshape(1, 64)
    betap = jnp.pad(beta, (0, 14)).reshape(1, 64)
    fw1 = _pad2(fcw1, 64, 128)
    fb1 = jnp.pad(fcb1, (0, 98)).reshape(1, 128)
    fw2 = _pad2(fcw2, 128, 128)
    fb2 = jnp.pad(fcb2, (0, 108)).reshape(1, 128)
    fw3 = _pad2(fcw3, 128, 128)
    fb3 = jnp.pad(fcb3, (0, 126)).reshape(1, 128)

    src2 = src.reshape(E // C, C)
    dst2 = dst.reshape(E // C, C)
    ew2 = edge_weigth.reshape(E // C, C)

    def scat(h, F):
        zeros = jnp.zeros((N, F), jnp.float32)
        p = _sc_scatter(F)(h, src2, dst2, ew2, zeros)
        return p.reshape(2, N, F)

    h1 = _mm_first(x, W1)
    p1 = scat(h1, 16)
    h2 = _fuse_layer(p1, b1.reshape(1, 16), W2)
    p2 = scat(h2, 32)
    h3 = _fuse_layer(p2, b2.reshape(1, 32), W3)
    p3 = scat(h3, 64)
    h4 = _fuse_layer(p3, b3.reshape(1, 64), W4p)
    p4 = scat(h4, 64)
    conv, stats = _final_conv(p4, b4p)
    pooled = _bn_pool(conv, stats, gammap, betap, batch.reshape(N // BR, 1, BR))
    out = _mlp(pooled, fw1, fb1, fw2, fb2, fw3, fb3)
    return out[:, :2]


# speedup vs baseline: 9.4267x; 1.1801x over previous
import functools

import jax
import jax.numpy as jnp
from jax import lax
from jax.experimental import pallas as pl
from jax.experimental.pallas import tpu as pltpu
from jax.experimental.pallas import tpu_sc as plsc

N = 10000
E = 320000
G = 256

NC = 2
NS = 16
NW = NC * NS
EPW = E // NW
C = 80
NCHUNK = EPW // C
K = 5
NSUPER = NCHUNK // K
NPAIR = NSUPER // 2
RPT = 624
RTAIL = N - NS * RPT

BR = 1000


def _leaky(t):
    return jnp.maximum(t, 0.01 * t)


@functools.cache
def _sc_scatter(F):
    mesh = plsc.VectorSubcoreMesh(core_axis_name="c", subcore_axis_name="s")

    @functools.partial(
        pl.kernel,
        out_type=jax.ShapeDtypeStruct((2 * N, F), jnp.float32),
        mesh=mesh,
        scratch_types=(
            [pltpu.VMEM((NCHUNK, C), jnp.int32),
             pltpu.VMEM((NCHUNK, C), jnp.int32),
             pltpu.VMEM((NCHUNK, C), jnp.float32)]
            + [pltpu.VMEM((C, F), jnp.float32) for _ in range(2 * K)]
            + [pltpu.VMEM_SHARED((N, F), jnp.float32),
               pltpu.SemaphoreType.DMA,
               pltpu.SemaphoreType.DMA,
               pltpu.SemaphoreType.DMA,
               pltpu.SemaphoreType.DMA]
        ),
        compiler_params=pltpu.CompilerParams(use_tc_tiling_on_sc=False),
    )
    def scat(h_hbm, src_hbm, dst_hbm, ew_hbm, zero_hbm, out_hbm, *refs):
        src_v, dst_v, ew_v = refs[0], refs[1], refs[2]
        rows_a = refs[3:3 + K]
        rows_b = refs[3 + K:3 + 2 * K]
        acc_s = refs[3 + 2 * K]
        gsem_a, gsem_b, ssem_a, ssem_b = refs[4 + 2 * K:8 + 2 * K]
        cid = lax.axis_index("c")
        sid = lax.axis_index("s")
        wid = sid * NC + cid

        pltpu.sync_copy(zero_hbm.at[pl.ds(sid * RPT, RPT)],
                        acc_s.at[pl.ds(sid * RPT, RPT)])

        @pl.when(sid == NS - 1)
        def _():
            pltpu.sync_copy(zero_hbm.at[pl.ds(NS * RPT, RTAIL)],
                            acc_s.at[pl.ds(NS * RPT, RTAIL)])

        cbase0 = wid * NCHUNK
        pltpu.sync_copy(src_hbm.at[pl.ds(cbase0, NCHUNK)], src_v)
        pltpu.sync_copy(dst_hbm.at[pl.ds(cbase0, NCHUNK)], dst_v)
        pltpu.sync_copy(ew_hbm.at[pl.ds(cbase0, NCHUNK)], ew_v)
        plsc.subcore_barrier()

        def fire_g(bufs, sem, cb):
            for j in range(K):
                pltpu.async_copy(h_hbm.at[src_v.at[cb + j]], bufs[j], sem)

        def drain_g(bufs, sem):
            for j in range(K):
                pltpu.make_async_copy(h_hbm.at[src_v.at[0]], bufs[j],
                                      sem).wait()

        def fire_s(bufs, sem, cb):
            for j in range(K):
                pltpu.async_copy(bufs[j], acc_s.at[dst_v.at[cb + j]], sem,
                                 add=True)

        def drain_s(bufs, sem):
            for j in range(K):
                pltpu.make_async_copy(bufs[j], acc_s.at[dst_v.at[0]],
                                      sem).wait()

        def compute(bufs, cb):
            for j in range(K):
                def group(g, carry2, j=j):
                    ew16 = ew_v[cb + j, pl.ds(g * 16, 16)]
                    for e in range(16):
                        w = ew16.at[jnp.full((16,), e, jnp.int32)].get(
                            mode="promise_in_bounds")
                        r = g * 16 + e
                        for f in range(F // 16):
                            v = bufs[j][r, pl.ds(f * 16, 16)]
                            bufs[j][r, pl.ds(f * 16, 16)] = v * w
                    return carry2

                lax.fori_loop(0, C // 16, group, 0)

        fire_g(rows_a, gsem_a, 0)

        def pair(i, carry):
            a = 2 * i * K
            b = a + K

            @pl.when(i > 0)
            def _():
                drain_s(rows_b, ssem_b)

            fire_g(rows_b, gsem_b, b)
            drain_g(rows_a, gsem_a)
            compute(rows_a, a)
            fire_s(rows_a, ssem_a, a)
            drain_s(rows_a, ssem_a)
            fire_g(rows_a, gsem_a, a + 2 * K)
            drain_g(rows_b, gsem_b)
            compute(rows_b, b)
            fire_s(rows_b, ssem_b, b)
            return carry

        lax.fori_loop(0, NPAIR, pair, 0)
        eb = 2 * NPAIR * K
        drain_s(rows_b, ssem_b)
        drain_g(rows_a, gsem_a)
        compute(rows_a, eb)
        fire_s(rows_a, ssem_a, eb)
        drain_s(rows_a, ssem_a)
        plsc.subcore_barrier()
        pltpu.sync_copy(acc_s.at[pl.ds(sid * RPT, RPT)],
                        out_hbm.at[pl.ds(cid * N + sid * RPT, RPT)])

        @pl.when(sid == NS - 1)
        def _():
            pltpu.sync_copy(acc_s.at[pl.ds(NS * RPT, RTAIL)],
                            out_hbm.at[pl.ds(cid * N + NS * RPT, RTAIL)])

    return scat


def _mm_first(x, W):
    def body(x_ref, w_ref, o_ref):
        o_ref[...] = jnp.dot(x_ref[...], w_ref[...],
                             preferred_element_type=jnp.float32)

    Fi, Fo = W.shape
    return pl.pallas_call(
        body,
        grid=(N // BR,),
        in_specs=[pl.BlockSpec((BR, Fi), lambda i: (i, 0)),
                  pl.BlockSpec((Fi, Fo), lambda i: (0, 0))],
        out_specs=pl.BlockSpec((BR, Fo), lambda i: (i, 0)),
        out_shape=jax.ShapeDtypeStruct((N, Fo), jnp.float32),
    )(x, W)


def _fuse_layer(p, b, W):
    def body(p_ref, b_ref, w_ref, o_ref):
        s = p_ref[0] + p_ref[1] + b_ref[...]
        o_ref[...] = jnp.dot(_leaky(s), w_ref[...],
                             preferred_element_type=jnp.float32)

    F = p.shape[-1]
    Fo = W.shape[1]
    return pl.pallas_call(
        body,
        grid=(N // BR,),
        in_specs=[pl.BlockSpec((2, BR, F), lambda i: (0, i, 0)),
                  pl.BlockSpec((1, F), lambda i: (0, 0)),
                  pl.BlockSpec((F, Fo), lambda i: (0, 0))],
        out_specs=pl.BlockSpec((BR, Fo), lambda i: (i, 0)),
        out_shape=jax.ShapeDtypeStruct((N, Fo), jnp.float32),
    )(p, b, W)


def _final_conv(p, b):
    def body(p_ref, b_ref, conv_ref, st_ref):
        s = p_ref[0] + p_ref[1] + b_ref[...]
        conv_ref[...] = s

        @pl.when(pl.program_id(0) == 0)
        def _():
            st_ref[...] = jnp.zeros_like(st_ref)

        st_ref[0:1, :] = st_ref[0:1, :] + jnp.sum(s, axis=0, keepdims=True)
        st_ref[1:2, :] = st_ref[1:2, :] + jnp.sum(s * s, axis=0,
                                                  keepdims=True)

    F = p.shape[-1]
    return pl.pallas_call(
        body,
        grid=(N // BR,),
        in_specs=[pl.BlockSpec((2, BR, F), lambda i: (0, i, 0)),
                  pl.BlockSpec((1, F), lambda i: (0, 0))],
        out_specs=[pl.BlockSpec((BR, F), lambda i: (i, 0)),
                   pl.BlockSpec((8, F), lambda i: (0, 0))],
        out_shape=[jax.ShapeDtypeStruct((N, F), jnp.float32),
                   jax.ShapeDtypeStruct((8, F), jnp.float32)],
    )(p, b)


def _bn_pool(conv, stats, gamma, beta, batch3d):
    def body(c_ref, st_ref, g_ref, b_ref, bt_ref, o_ref):
        mean = st_ref[0:1, :] * (1.0 / N)
        var = st_ref[1:2, :] * (1.0 / N) - mean * mean
        inv = lax.rsqrt(var + 1e-5)
        s = (c_ref[...] - mean) * inv * g_ref[...] + b_ref[...]
        s = _leaky(s)
        sel = (bt_ref[0] ==
               lax.broadcasted_iota(jnp.int32, (G, 1), 0)).astype(jnp.float32)
        part = jnp.dot(sel, s, preferred_element_type=jnp.float32)

        @pl.when(pl.program_id(0) == 0)
        def _():
            o_ref[...] = jnp.zeros_like(o_ref)

        o_ref[...] = o_ref[...] + part

    F = conv.shape[-1]
    return pl.pallas_call(
        body,
        grid=(N // BR,),
        in_specs=[pl.BlockSpec((BR, F), lambda i: (i, 0)),
                  pl.BlockSpec((8, F), lambda i: (0, 0)),
                  pl.BlockSpec((1, F), lambda i: (0, 0)),
                  pl.BlockSpec((1, F), lambda i: (0, 0)),
                  pl.BlockSpec((1, 1, BR), lambda i: (i, 0, 0))],
        out_specs=pl.BlockSpec((G, F), lambda i: (0, 0)),
        out_shape=jax.ShapeDtypeStruct((G, F), jnp.float32),
    )(conv, stats, gamma, beta, batch3d)


def _mlp(pooled, w1, b1, w2, b2, w3, b3):
    def body(p_ref, w1r, b1r, w2r, b2r, w3r, b3r, o_ref):
        a = _leaky(jnp.dot(p_ref[...], w1r[...],
                           preferred_element_type=jnp.float32) + b1r[...])
        a = _leaky(jnp.dot(a, w2r[...],
                           preferred_element_type=jnp.float32) + b2r[...])
        a = _leaky(jnp.dot(a, w3r[...],
                           preferred_element_type=jnp.float32) + b3r[...])
        o_ref[...] = a

    H = w1.shape[1]
    return pl.pallas_call(
        body,
        in_specs=[pl.BlockSpec(pooled.shape, lambda: (0, 0)),
                  pl.BlockSpec(w1.shape, lambda: (0, 0)),
                  pl.BlockSpec(b1.shape, lambda: (0, 0)),
                  pl.BlockSpec(w2.shape, lambda: (0, 0)),
                  pl.BlockSpec(b2.shape, lambda: (0, 0)),
                  pl.BlockSpec(w3.shape, lambda: (0, 0)),
                  pl.BlockSpec(b3.shape, lambda: (0, 0))],
        out_specs=pl.BlockSpec((G, H), lambda: (0, 0)),
        out_shape=jax.ShapeDtypeStruct((G, H), jnp.float32),
    )(pooled, w1, b1, w2, b2, w3, b3)


def _pad2(a, r, c):
    return jnp.pad(a, ((0, r - a.shape[0]), (0, c - a.shape[1])))


def kernel(x, edge_index, edge_weigth, batch, W1, b1, W2, b2, W3, b3, W4, b4,
           gamma, beta, fcw1, fcb1, fcw2, fcb2, fcw3, fcb3):
    src = edge_index[0]
    dst = edge_index[1]

    W4p = _pad2(W4, 64, 64)
    b4p = jnp.pad(b4, (0, 14)).reshape(1, 64)
    gammap = jnp.pad(gamma, (0, 14)).reshape(1, 64)
    betap = jnp.pad(beta, (0, 14)).reshape(1, 64)
    fw1 = _pad2(fcw1, 64, 128)
    fb1 = jnp.pad(fcb1, (0, 98)).reshape(1, 128)
    fw2 = _pad2(fcw2, 128, 128)
    fb2 = jnp.pad(fcb2, (0, 108)).reshape(1, 128)
    fw3 = _pad2(fcw3, 128, 128)
    fb3 = jnp.pad(fcb3, (0, 126)).reshape(1, 128)

    src2 = src.reshape(E // C, C)
    dst2 = dst.reshape(E // C, C)
    ew2 = edge_weigth.reshape(E // C, C)

    def scat(h, F):
        zeros = jnp.zeros((N, F), jnp.float32)
        p = _sc_scatter(F)(h, src2, dst2, ew2, zeros)
        return p.reshape(2, N, F)

    h1 = _mm_first(x, W1)
    p1 = scat(h1, 16)
    h2 = _fuse_layer(p1, b1.reshape(1, 16), W2)
    p2 = scat(h2, 32)
    h3 = _fuse_layer(p2, b2.reshape(1, 32), W3)
    p3 = scat(h3, 64)
    h4 = _fuse_layer(p3, b3.reshape(1, 64), W4p)
    p4 = scat(h4, 64)
    conv, stats = _final_conv(p4, b4p)
    pooled = _bn_pool(conv, stats, gammap, betap, batch.reshape(N // BR, 1, BR))
    out = _mlp(pooled, fw1, fb1, fw2, fb2, fw3, fb3)
    return out[:, :2]


# speedup vs baseline: 15.0926x; 1.6010x over previous
import functools

import jax
import jax.numpy as jnp
from jax import lax
from jax.experimental import pallas as pl
from jax.experimental.pallas import tpu as pltpu
from jax.experimental.pallas import tpu_sc as plsc

N = 10000
E = 320000
G = 256

NC = 2
NS = 16
NW = NC * NS
EPW = E // NW
C = 80
NCHUNK = EPW // C
K = 5
NSUPER = NCHUNK // K
NPAIR = NSUPER // 2
RPT = 624
RTAIL = N - NS * RPT

BR = 1000


def _leaky(t):
    return jnp.maximum(t, 0.01 * t)


@functools.cache
def _sc_scatter(F):
    mesh = plsc.VectorSubcoreMesh(core_axis_name="c", subcore_axis_name="s")

    @functools.partial(
        pl.kernel,
        out_type=jax.ShapeDtypeStruct((2 * N, F), jnp.float32),
        mesh=mesh,
        scratch_types=(
            [pltpu.VMEM((NCHUNK, C), jnp.int32),
             pltpu.VMEM((NCHUNK, C), jnp.int32),
             pltpu.VMEM((NCHUNK, C), jnp.float32)]
            + [pltpu.VMEM((C, F), jnp.float32) for _ in range(2 * K)]
            + [pltpu.VMEM_SHARED((N, F), jnp.float32),
               pltpu.SemaphoreType.DMA,
               pltpu.SemaphoreType.DMA,
               pltpu.SemaphoreType.DMA,
               pltpu.SemaphoreType.DMA]
        ),
        compiler_params=pltpu.CompilerParams(use_tc_tiling_on_sc=False),
    )
    def scat(h_hbm, src_hbm, dst_hbm, ew_hbm, zero_hbm, out_hbm, *refs):
        src_v, dst_v, ew_v = refs[0], refs[1], refs[2]
        rows_a = refs[3:3 + K]
        rows_b = refs[3 + K:3 + 2 * K]
        acc_s = refs[3 + 2 * K]
        gsem_a, gsem_b, ssem_a, ssem_b = refs[4 + 2 * K:8 + 2 * K]
        cid = lax.axis_index("c")
        sid = lax.axis_index("s")
        wid = sid * NC + cid

        pltpu.sync_copy(zero_hbm.at[pl.ds(sid * RPT, RPT)],
                        acc_s.at[pl.ds(sid * RPT, RPT)])

        @pl.when(sid == NS - 1)
        def _():
            pltpu.sync_copy(zero_hbm.at[pl.ds(NS * RPT, RTAIL)],
                            acc_s.at[pl.ds(NS * RPT, RTAIL)])

        cbase0 = wid * NCHUNK
        pltpu.sync_copy(src_hbm.at[pl.ds(cbase0, NCHUNK)], src_v)
        pltpu.sync_copy(dst_hbm.at[pl.ds(cbase0, NCHUNK)], dst_v)
        pltpu.sync_copy(ew_hbm.at[pl.ds(cbase0, NCHUNK)], ew_v)
        plsc.subcore_barrier()

        def fire_g(bufs, sem, cb):
            for j in range(K):
                pltpu.async_copy(h_hbm.at[src_v.at[cb + j]], bufs[j], sem)

        def drain_g(bufs, sem):
            for j in range(K):
                pltpu.make_async_copy(h_hbm.at[src_v.at[0]], bufs[j],
                                      sem).wait()

        def fire_s(bufs, sem, cb):
            for j in range(K):
                pltpu.async_copy(bufs[j], acc_s.at[dst_v.at[cb + j]], sem,
                                 add=True)

        def drain_s(bufs, sem):
            for j in range(K):
                pltpu.make_async_copy(bufs[j], acc_s.at[dst_v.at[0]],
                                      sem).wait()

        def compute(bufs, cb):
            for j in range(K):
                @plsc.parallel_loop(0, C, step=16, unroll=2)
                def group(gbase, j=j):
                    ew16 = ew_v[cb + j, pl.ds(gbase, 16)]
                    for e in range(16):
                        w = ew16.at[jnp.full((16,), e, jnp.int32)].get(
                            mode="promise_in_bounds")
                        for f in range(F // 16):
                            v = bufs[j][gbase + e, pl.ds(f * 16, 16)]
                            bufs[j][gbase + e, pl.ds(f * 16, 16)] = v * w

        fire_g(rows_a, gsem_a, 0)

        def pair(i, carry):
            a = 2 * i * K
            b = a + K

            @pl.when(i > 0)
            def _():
                drain_s(rows_b, ssem_b)

            fire_g(rows_b, gsem_b, b)
            drain_g(rows_a, gsem_a)
            compute(rows_a, a)
            fire_s(rows_a, ssem_a, a)
            drain_s(rows_a, ssem_a)
            fire_g(rows_a, gsem_a, a + 2 * K)
            drain_g(rows_b, gsem_b)
            compute(rows_b, b)
            fire_s(rows_b, ssem_b, b)
            return carry

        lax.fori_loop(0, NPAIR, pair, 0)
        eb = 2 * NPAIR * K
        drain_s(rows_b, ssem_b)
        drain_g(rows_a, gsem_a)
        compute(rows_a, eb)
        fire_s(rows_a, ssem_a, eb)
        drain_s(rows_a, ssem_a)
        plsc.subcore_barrier()
        pltpu.sync_copy(acc_s.at[pl.ds(sid * RPT, RPT)],
                        out_hbm.at[pl.ds(cid * N + sid * RPT, RPT)])

        @pl.when(sid == NS - 1)
        def _():
            pltpu.sync_copy(acc_s.at[pl.ds(NS * RPT, RTAIL)],
                            out_hbm.at[pl.ds(cid * N + NS * RPT, RTAIL)])

    return scat


def _mm_first(x, W):
    def body(x_ref, w_ref, o_ref):
        o_ref[...] = jnp.dot(x_ref[...], w_ref[...],
                             preferred_element_type=jnp.float32)

    Fi, Fo = W.shape
    return pl.pallas_call(
        body,
        grid=(N // BR,),
        in_specs=[pl.BlockSpec((BR, Fi), lambda i: (i, 0)),
                  pl.BlockSpec((Fi, Fo), lambda i: (0, 0))],
        out_specs=pl.BlockSpec((BR, Fo), lambda i: (i, 0)),
        out_shape=jax.ShapeDtypeStruct((N, Fo), jnp.float32),
    )(x, W)


def _fuse_layer(p, b, W):
    def body(p_ref, b_ref, w_ref, o_ref):
        s = p_ref[0] + p_ref[1] + b_ref[...]
        o_ref[...] = jnp.dot(_leaky(s), w_ref[...],
                             preferred_element_type=jnp.float32)

    F = p.shape[-1]
    Fo = W.shape[1]
    return pl.pallas_call(
        body,
        grid=(N // BR,),
        in_specs=[pl.BlockSpec((2, BR, F), lambda i: (0, i, 0)),
                  pl.BlockSpec((1, F), lambda i: (0, 0)),
                  pl.BlockSpec((F, Fo), lambda i: (0, 0))],
        out_specs=pl.BlockSpec((BR, Fo), lambda i: (i, 0)),
        out_shape=jax.ShapeDtypeStruct((N, Fo), jnp.float32),
    )(p, b, W)


def _final_conv(p, b):
    def body(p_ref, b_ref, conv_ref, st_ref):
        s = p_ref[0] + p_ref[1] + b_ref[...]
        conv_ref[...] = s

        @pl.when(pl.program_id(0) == 0)
        def _():
            st_ref[...] = jnp.zeros_like(st_ref)

        st_ref[0:1, :] = st_ref[0:1, :] + jnp.sum(s, axis=0, keepdims=True)
        st_ref[1:2, :] = st_ref[1:2, :] + jnp.sum(s * s, axis=0,
                                                  keepdims=True)

    F = p.shape[-1]
    return pl.pallas_call(
        body,
        grid=(N // BR,),
        in_specs=[pl.BlockSpec((2, BR, F), lambda i: (0, i, 0)),
                  pl.BlockSpec((1, F), lambda i: (0, 0))],
        out_specs=[pl.BlockSpec((BR, F), lambda i: (i, 0)),
                   pl.BlockSpec((8, F), lambda i: (0, 0))],
        out_shape=[jax.ShapeDtypeStruct((N, F), jnp.float32),
                   jax.ShapeDtypeStruct((8, F), jnp.float32)],
    )(p, b)


def _bn_pool(conv, stats, gamma, beta, batch3d):
    def body(c_ref, st_ref, g_ref, b_ref, bt_ref, o_ref):
        mean = st_ref[0:1, :] * (1.0 / N)
        var = st_ref[1:2, :] * (1.0 / N) - mean * mean
        inv = lax.rsqrt(var + 1e-5)
        s = (c_ref[...] - mean) * inv * g_ref[...] + b_ref[...]
        s = _leaky(s)
        sel = (bt_ref[0] ==
               lax.broadcasted_iota(jnp.int32, (G, 1), 0)).astype(jnp.float32)
        part = jnp.dot(sel, s, preferred_element_type=jnp.float32)

        @pl.when(pl.program_id(0) == 0)
        def _():
            o_ref[...] = jnp.zeros_like(o_ref)

        o_ref[...] = o_ref[...] + part

    F = conv.shape[-1]
    return pl.pallas_call(
        body,
        grid=(N // BR,),
        in_specs=[pl.BlockSpec((BR, F), lambda i: (i, 0)),
                  pl.BlockSpec((8, F), lambda i: (0, 0)),
                  pl.BlockSpec((1, F), lambda i: (0, 0)),
                  pl.BlockSpec((1, F), lambda i: (0, 0)),
                  pl.BlockSpec((1, 1, BR), lambda i: (i, 0, 0))],
        out_specs=pl.BlockSpec((G, F), lambda i: (0, 0)),
        out_shape=jax.ShapeDtypeStruct((G, F), jnp.float32),
    )(conv, stats, gamma, beta, batch3d)


def _mlp(pooled, w1, b1, w2, b2, w3, b3):
    def body(p_ref, w1r, b1r, w2r, b2r, w3r, b3r, o_ref):
        a = _leaky(jnp.dot(p_ref[...], w1r[...],
                           preferred_element_type=jnp.float32) + b1r[...])
        a = _leaky(jnp.dot(a, w2r[...],
                           preferred_element_type=jnp.float32) + b2r[...])
        a = _leaky(jnp.dot(a, w3r[...],
                           preferred_element_type=jnp.float32) + b3r[...])
        o_ref[...] = a

    H = w1.shape[1]
    return pl.pallas_call(
        body,
        in_specs=[pl.BlockSpec(pooled.shape, lambda: (0, 0)),
                  pl.BlockSpec(w1.shape, lambda: (0, 0)),
                  pl.BlockSpec(b1.shape, lambda: (0, 0)),
                  pl.BlockSpec(w2.shape, lambda: (0, 0)),
                  pl.BlockSpec(b2.shape, lambda: (0, 0)),
                  pl.BlockSpec(w3.shape, lambda: (0, 0)),
                  pl.BlockSpec(b3.shape, lambda: (0, 0))],
        out_specs=pl.BlockSpec((G, H), lambda: (0, 0)),
        out_shape=jax.ShapeDtypeStruct((G, H), jnp.float32),
    )(pooled, w1, b1, w2, b2, w3, b3)


def _pad2(a, r, c):
    return jnp.pad(a, ((0, r - a.shape[0]), (0, c - a.shape[1])))


def kernel(x, edge_index, edge_weigth, batch, W1, b1, W2, b2, W3, b3, W4, b4,
           gamma, beta, fcw1, fcb1, fcw2, fcb2, fcw3, fcb3):
    src = edge_index[0]
    dst = edge_index[1]

    W4p = _pad2(W4, 64, 64)
    b4p = jnp.pad(b4, (0, 14)).reshape(1, 64)
    gammap = jnp.pad(gamma, (0, 14)).reshape(1, 64)
    betap = jnp.pad(beta, (0, 14)).reshape(1, 64)
    fw1 = _pad2(fcw1, 64, 128)
    fb1 = jnp.pad(fcb1, (0, 98)).reshape(1, 128)
    fw2 = _pad2(fcw2, 128, 128)
    fb2 = jnp.pad(fcb2, (0, 108)).reshape(1, 128)
    fw3 = _pad2(fcw3, 128, 128)
    fb3 = jnp.pad(fcb3, (0, 126)).reshape(1, 128)

    src2 = src.reshape(E // C, C)
    dst2 = dst.reshape(E // C, C)
    ew2 = edge_weigth.reshape(E // C, C)

    def scat(h, F):
        zeros = jnp.zeros((N, F), jnp.float32)
        p = _sc_scatter(F)(h, src2, dst2, ew2, zeros)
        return p.reshape(2, N, F)

    h1 = _mm_first(x, W1)
    p1 = scat(h1, 16)
    h2 = _fuse_layer(p1, b1.reshape(1, 16), W2)
    p2 = scat(h2, 32)
    h3 = _fuse_layer(p2, b2.reshape(1, 32), W3)
    p3 = scat(h3, 64)
    h4 = _fuse_layer(p3, b3.reshape(1, 64), W4p)
    p4 = scat(h4, 64)
    conv, stats = _final_conv(p4, b4p)
    pooled = _bn_pool(conv, stats, gammap, betap, batch.reshape(N // BR, 1, BR))
    out = _mlp(pooled, fw1, fb1, fw2, fb2, fw3, fb3)
    return out[:, :2]
